# Initial kernel scaffold; baseline (speedup 1.0000x reference)
import jax
import jax.numpy as jnp
from jax.experimental import pallas as pl


def kernel(hidden_states, gate_w, gate2_w, W1, W2, W3):
    raise NotImplementedError("write your pallas kernel here")


# speedup vs baseline: 1.3779x; 1.3779x over previous
import functools

import jax
import jax.numpy as jnp
from jax.experimental import pallas as pl
from jax.experimental.pallas import tpu as pltpu

E = 8
NE = 10
EP = 16
TOPK = 2


def _router_body(x_ref, wg_ref, logits_ref, we_ref):
    x = x_ref[...]
    wg = wg_ref[...]
    logits = jax.lax.dot_general(
        x, wg, (((1,), (1,)), ((), ())), preferred_element_type=jnp.float32)
    s = logits.shape[0]
    col = jax.lax.broadcasted_iota(jnp.int32, (s, EP), 1)
    lm = jnp.where(col < NE, logits, jnp.float32(-1e30))
    m = jnp.max(lm, axis=1, keepdims=True)
    p = jnp.exp(lm - m)
    probs = p / jnp.sum(p, axis=1, keepdims=True)
    v1 = jnp.max(probs, axis=1, keepdims=True)
    i1 = jnp.min(jnp.where(probs == v1, col, EP + 1), axis=1, keepdims=True)
    probs2 = jnp.where(col == i1, jnp.float32(-1.0), probs)
    v2 = jnp.max(probs2, axis=1, keepdims=True)
    i2 = jnp.min(jnp.where(probs2 == v2, col, EP + 1), axis=1, keepdims=True)
    real1 = (i1 < E).astype(jnp.float32)
    real2 = (i2 < E).astype(jnp.float32)
    ssum = v1 * real1 + v2 * real2
    denom = jnp.where(ssum == 0.0, jnp.float32(1.0), ssum)
    w1 = real1 * v1 / denom
    w2 = real2 * v2 / denom
    oh1 = (col == i1).astype(jnp.float32)
    oh2 = (col == i2).astype(jnp.float32)
    we_ref[...] = w1 * oh1 + w2 * oh2
    logits_ref[...] = logits


def _ffn_body(x_ref, w1_ref, w3_ref, w2_ref, wet_ref, out_ref):
    e = pl.program_id(0)
    kf = pl.program_id(1)
    x = x_ref[...]
    w1c = w1_ref[0]
    w3c = w3_ref[0]
    w2c = w2_ref[0]
    a = jax.lax.dot_general(x, w1c, (((1,), (1,)), ((), ())),
                            preferred_element_type=jnp.float32)
    b = jax.lax.dot_general(x, w3c, (((1,), (1,)), ((), ())),
                            preferred_element_type=jnp.float32)
    h = (a * jax.nn.sigmoid(a)) * b
    part = jax.lax.dot_general(h, w2c, (((1,), (1,)), ((), ())),
                               preferred_element_type=jnp.float32)
    wcol = wet_ref[0, 0, :]
    contrib = part * wcol[:, None]

    @pl.when(jnp.logical_and(e == 0, kf == 0))
    def _():
        out_ref[...] = contrib

    @pl.when(jnp.logical_or(e > 0, kf > 0))
    def _():
        out_ref[...] = out_ref[...] + contrib


@functools.partial(jax.jit, static_argnames=())
def kernel(hidden_states, gate_w, gate2_w, W1, W2, W3):
    b, s, h = hidden_states.shape
    ff = W1.shape[1]
    x = hidden_states.reshape(s, h)
    wg = jnp.concatenate([gate_w, gate2_w], axis=0)
    wg = jnp.pad(wg, ((0, EP - NE), (0, 0)))

    logits_p, we = pl.pallas_call(
        _router_body,
        out_shape=(
            jax.ShapeDtypeStruct((s, EP), jnp.float32),
            jax.ShapeDtypeStruct((s, EP), jnp.float32),
        ),
    )(x, wg)

    wet = we.T.reshape(EP, 1, s)

    fc = ff // 4
    kfn = ff // fc
    out = pl.pallas_call(
        _ffn_body,
        grid=(E, kfn),
        in_specs=[
            pl.BlockSpec((s, h), lambda e, kf: (0, 0)),
            pl.BlockSpec((1, fc, h), lambda e, kf: (e, kf, 0)),
            pl.BlockSpec((1, fc, h), lambda e, kf: (e, kf, 0)),
            pl.BlockSpec((1, h, fc), lambda e, kf: (e, 0, kf)),
            pl.BlockSpec((1, 1, s), lambda e, kf: (e, 0, 0)),
        ],
        out_specs=pl.BlockSpec((s, h), lambda e, kf: (0, 0)),
        out_shape=jax.ShapeDtypeStruct((s, h), jnp.float32),
    )(x, W1, W3, W2, wet)

    return out.reshape(b, s, h), logits_p[:, :NE]


# speedup vs baseline: 2.4793x; 1.7993x over previous
import functools

import jax
import jax.numpy as jnp
from jax import lax
from jax.experimental import pallas as pl
from jax.experimental.pallas import tpu as pltpu
from jax.experimental.pallas import tpu_sc as plsc

E = 8
NE = 10
EP = 16
S = 2048
H = 1024
FF = 4096
T = 512
G = 16
FC = 1024
KFN = FF // FC
NROWS = G * T
TRASH = NROWS - 1
NC, NS = 2, 16
NW = NC * NS
TPW = S // NW
CCH = 16


def _router_body(x_ref, wg_ref, logits_ref, dst0_ref, dst1_ref,
                 wb0_ref, wb1_ref, cnts_ref):
    x = x_ref[...]
    wg = wg_ref[...]
    logits = lax.dot_general(x, wg, (((1,), (1,)), ((), ())),
                             preferred_element_type=jnp.float32)
    col = lax.broadcasted_iota(jnp.int32, (S, EP), 1)
    lm = jnp.where(col < NE, logits, jnp.float32(-1e30))
    m = jnp.max(lm, axis=1, keepdims=True)
    p = jnp.exp(lm - m)
    probs = p / jnp.sum(p, axis=1, keepdims=True)
    v1 = jnp.max(probs, axis=1, keepdims=True)
    i1 = jnp.min(jnp.where(probs == v1, col, EP + 1), axis=1, keepdims=True)
    probs2 = jnp.where(col == i1, jnp.float32(-1.0), probs)
    v2 = jnp.max(probs2, axis=1, keepdims=True)
    i2 = jnp.min(jnp.where(probs2 == v2, col, EP + 1), axis=1, keepdims=True)
    real1 = (i1 < E).astype(jnp.float32)
    real2 = (i2 < E).astype(jnp.float32)
    ssum = v1 * real1 + v2 * real2
    denom = jnp.where(ssum == 0.0, jnp.float32(1.0), ssum)
    w1 = real1 * v1 / denom
    w2 = real2 * v2 / denom
    logits_ref[...] = logits
    wb0_ref[...] = jnp.broadcast_to(w1, (S, EP))
    wb1_ref[...] = jnp.broadcast_to(w2, (S, EP))

    colw = lax.broadcasted_iota(jnp.int32, (S, 128), 1)
    oh1 = ((colw == i1) & (colw < E)).astype(jnp.float32)
    oh2 = ((colw == i2) & (colw < E)).astype(jnp.float32)
    cnt = oh1 + oh2
    ri = lax.broadcasted_iota(jnp.int32, (128, 128), 0)
    ci = lax.broadcasted_iota(jnp.int32, (128, 128), 1)
    lstrict = (ci < ri).astype(jnp.float32)
    ustrict = (ri < ci).astype(jnp.float32)
    nch = S // 128
    within = []
    tots = []
    for c in range(nch):
        seg = cnt[c * 128:(c + 1) * 128, :]
        within.append(lax.dot_general(lstrict, seg, (((1,), (0,)), ((), ())),
                                      preferred_element_type=jnp.float32))
        tots.append(jnp.sum(seg, axis=0, keepdims=True))
    within = jnp.concatenate(within, axis=0)
    tots = jnp.concatenate(tots, axis=0)
    ri16 = lax.broadcasted_iota(jnp.int32, (nch, nch), 0)
    ci16 = lax.broadcasted_iota(jnp.int32, (nch, nch), 1)
    l16 = (ci16 < ri16).astype(jnp.float32)
    pref = lax.dot_general(l16, tots, (((1,), (0,)), ((), ())),
                           preferred_element_type=jnp.float32)
    pref_full = jnp.broadcast_to(pref[:, None, :], (nch, 128, 128)).reshape(S, 128)
    rank = within + pref_full
    counts = jnp.sum(tots, axis=0, keepdims=True)
    ci32 = counts.astype(jnp.int32)
    pad = ((ci32 + (T - 1)) // T) * T
    starts = lax.dot_general(pad.astype(jnp.float32), ustrict,
                             (((1,), (0,)), ((), ())),
                             preferred_element_type=jnp.float32)
    pos = starts + rank
    sel1 = jnp.sum(oh1 * pos, axis=1, keepdims=True)
    sel2 = jnp.sum(oh2 * pos, axis=1, keepdims=True)
    dst0_ref[...] = jnp.where(real1 > 0, sel1, jnp.float32(TRASH)).astype(jnp.int32)
    dst1_ref[...] = jnp.where(real2 > 0, sel2, jnp.float32(TRASH)).astype(jnp.int32)
    cnts_ref[...] = jnp.broadcast_to(counts, (8, 128))


_router = pl.pallas_call(
    _router_body,
    out_shape=(
        jax.ShapeDtypeStruct((S, EP), jnp.float32),
        jax.ShapeDtypeStruct((S, 1), jnp.int32),
        jax.ShapeDtypeStruct((S, 1), jnp.int32),
        jax.ShapeDtypeStruct((S, EP), jnp.float32),
        jax.ShapeDtypeStruct((S, EP), jnp.float32),
        jax.ShapeDtypeStruct((8, 128), jnp.float32),
    ),
)

_sc_cache = {}


def _get_dispatch():
    if "dispatch" in _sc_cache:
        return _sc_cache["dispatch"]
    mesh = plsc.VectorSubcoreMesh(
        core_axis_name="c", subcore_axis_name="s", num_cores=NC, num_subcores=NS)

    @functools.partial(
        pl.kernel,
        out_type=jax.ShapeDtypeStruct((NROWS, H), jnp.float32),
        mesh=mesh,
        scratch_types=[
            pltpu.VMEM((TPW,), jnp.int32),
            pltpu.VMEM((TPW,), jnp.int32),
            pltpu.VMEM((TPW, H), jnp.float32),
            pltpu.SemaphoreType.DMA,
        ],
    )
    def _dispatch(x_hbm, dst0_hbm, dst1_hbm, xg_hbm, idx0_v, idx1_v, rows_v, sem):
        wid = lax.axis_index("s") * NC + lax.axis_index("c")
        base = wid * TPW
        pltpu.sync_copy(dst0_hbm.at[pl.ds(base, TPW)], idx0_v)
        pltpu.sync_copy(dst1_hbm.at[pl.ds(base, TPW)], idx1_v)
        pltpu.sync_copy(x_hbm.at[pl.ds(base, TPW)], rows_v)
        pltpu.async_copy(rows_v, xg_hbm.at[idx0_v], sem).wait()
        pltpu.async_copy(rows_v, xg_hbm.at[idx1_v], sem).wait()

    _sc_cache["dispatch"] = _dispatch
    return _dispatch


def _ffn_body(be_ref, bm_ref, na_ref, xg_ref, w1_ref, w3_ref, w2_ref, out_ref):
    b = pl.program_id(0)
    kf = pl.program_id(1)
    active = b < na_ref[0]

    @pl.when(active)
    def _():
        x = xg_ref[...]
        w1c = w1_ref[0]
        w3c = w3_ref[0]
        w2c = w2_ref[0]
        a = lax.dot_general(x, w1c, (((1,), (1,)), ((), ())),
                            preferred_element_type=jnp.float32)
        bb = lax.dot_general(x, w3c, (((1,), (1,)), ((), ())),
                             preferred_element_type=jnp.float32)
        hh = (a * jax.nn.sigmoid(a)) * bb
        part = lax.dot_general(hh, w2c, (((1,), (1,)), ((), ())),
                               preferred_element_type=jnp.float32)

        @pl.when(kf == 0)
        def _():
            out_ref[...] = part

        @pl.when(kf > 0)
        def _():
            out_ref[...] = out_ref[...] + part


def _kf_eff(b, kf, na):
    return jnp.where(b < na[0], kf, KFN - 1)


_ffn = pl.pallas_call(
    _ffn_body,
    grid_spec=pltpu.PrefetchScalarGridSpec(
        num_scalar_prefetch=3,
        grid=(G, KFN),
        in_specs=[
            pl.BlockSpec((T, H), lambda b, kf, be, bm, na: (bm[b], 0)),
            pl.BlockSpec((1, FC, H),
                         lambda b, kf, be, bm, na: (be[b], _kf_eff(b, kf, na), 0)),
            pl.BlockSpec((1, FC, H),
                         lambda b, kf, be, bm, na: (be[b], _kf_eff(b, kf, na), 0)),
            pl.BlockSpec((1, H, FC),
                         lambda b, kf, be, bm, na: (be[b], 0, _kf_eff(b, kf, na))),
        ],
        out_specs=pl.BlockSpec((T, H), lambda b, kf, be, bm, na: (bm[b], 0)),
    ),
    out_shape=jax.ShapeDtypeStruct((NROWS, H), jnp.float32),
)


def _get_combine():
    if "combine" in _sc_cache:
        return _sc_cache["combine"]
    mesh = plsc.VectorSubcoreMesh(
        core_axis_name="c", subcore_axis_name="s", num_cores=NC, num_subcores=NS)

    @functools.partial(
        pl.kernel,
        out_type=jax.ShapeDtypeStruct((S, H), jnp.float32),
        mesh=mesh,
        scratch_types=[
            pltpu.VMEM((CCH,), jnp.int32),
            pltpu.VMEM((CCH,), jnp.int32),
            pltpu.VMEM((CCH, EP), jnp.float32),
            pltpu.VMEM((CCH, EP), jnp.float32),
            pltpu.VMEM((CCH, H), jnp.float32),
            pltpu.VMEM((CCH, H), jnp.float32),
            pltpu.VMEM((CCH, H), jnp.float32),
            pltpu.SemaphoreType.DMA,
        ],
    )
    def _combine(y_hbm, dst0_hbm, dst1_hbm, wb0_hbm, wb1_hbm, out_hbm,
                 idx0_v, idx1_v, w0_v, w1_v, r0_v, r1_v, o_v, sem):
        wid = lax.axis_index("s") * NC + lax.axis_index("c")
        base = wid * TPW
        for ch in range(TPW // CCH):
            tb = base + ch * CCH
            pltpu.sync_copy(dst0_hbm.at[pl.ds(tb, CCH)], idx0_v)
            pltpu.sync_copy(dst1_hbm.at[pl.ds(tb, CCH)], idx1_v)
            pltpu.sync_copy(wb0_hbm.at[pl.ds(tb, CCH), :], w0_v)
            pltpu.sync_copy(wb1_hbm.at[pl.ds(tb, CCH), :], w1_v)
            pltpu.async_copy(y_hbm.at[idx0_v], r0_v, sem).wait()
            pltpu.async_copy(y_hbm.at[idx1_v], r1_v, sem).wait()

            def tok(i, carry):
                w0 = w0_v[i, :]
                w1 = w1_v[i, :]
                z0 = w0 == 0.0
                z1 = w1 == 0.0
                zero = jnp.zeros((16,), jnp.float32)
                for j in range(H // 16):
                    sl = pl.ds(j * 16, 16)
                    r0 = r0_v[i, sl]
                    r1 = r1_v[i, sl]
                    o_v[i, sl] = (jnp.where(z0, zero, w0 * r0)
                                  + jnp.where(z1, zero, w1 * r1))
                return carry

            lax.fori_loop(0, CCH, tok, 0)
            pltpu.sync_copy(o_v, out_hbm.at[pl.ds(tb, CCH), :])

    _sc_cache["combine"] = _combine
    return _combine


def kernel(hidden_states, gate_w, gate2_w, W1, W2, W3):
    b, s, h = hidden_states.shape
    x = hidden_states.reshape(s, h)
    wg = jnp.pad(jnp.concatenate([gate_w, gate2_w], axis=0),
                 ((0, EP - NE), (0, 0)))
    logits, dst0, dst1, wb0, wb1, cnts = _router(x, wg)
    dst0 = dst0.reshape(s)
    dst1 = dst1.reshape(s)
    c8 = cnts[0, :E].astype(jnp.int32)
    pad8 = ((c8 + T - 1) // T) * T
    ends = jnp.cumsum(pad8)
    na = ends[E - 1] // T
    bidx = jnp.arange(G, dtype=jnp.int32)
    be_full = jnp.minimum(
        jnp.sum((bidx[:, None] * T >= ends[None, :]).astype(jnp.int32), axis=1),
        E - 1)
    bm = jnp.minimum(bidx, jnp.maximum(na, 1) - 1).astype(jnp.int32)
    be = be_full[bm].astype(jnp.int32)
    na_arr = jnp.reshape(na, (1,)).astype(jnp.int32)

    xg = _get_dispatch()(x, dst0, dst1)
    y = _ffn(be, bm, na_arr, xg, W1, W3, W2)
    out = _get_combine()(y, dst0, dst1, wb0, wb1)
    return out.reshape(b, s, h), logits[:, :NE]
